# baseline (speedup 1.0000x reference)
import functools
import math

import jax
import jax.numpy as jnp
from jax import lax
from jax.experimental import pallas as pl
from jax.experimental.pallas import tpu as pltpu
from jax.experimental.pallas import tpu_sc as plsc

_EMB = 32
_SCALE = math.sqrt(_EMB)

_NC = 2
_NS = 16
_NW = _NC * _NS

_B0 = 4096
_B1 = 200
_CHUNK = 1024
_NQ = (_B0 // 128) * (_B1 // 8)
_CPW = _NQ // _NW


def _emb_kernel(tok_hbm, table_hbm, out_hbm, idx_v, rows_v, stage_v, gsem, wsem):
    wid = lax.axis_index("s") * _NC + lax.axis_index("c")
    q0 = wid * _CPW
    iota = lax.iota(jnp.int32, 16)

    def start_gather(qi, buf):
        pltpu.sync_copy(tok_hbm.at[q0 + qi], idx_v.at[buf])
        pltpu.async_copy(table_hbm.at[idx_v.at[buf]], rows_v.at[buf], gsem.at[buf])

    def wait_gather(buf):
        pltpu.make_async_copy(
            table_hbm.at[idx_v.at[buf]], rows_v.at[buf], gsem.at[buf]).wait()

    def drain_writes():
        for b in range(8):
            pltpu.make_async_copy(
                stage_v.at[b], out_hbm.at[0, :, pl.ds(0, _CHUNK)], wsem).wait()

    def do_chunk(qi, buf):
        q = q0 + qi
        a = lax.div(q, 32)
        c = lax.rem(q, 32)

        @pl.when(qi + 1 < _CPW)
        def _():
            start_gather(qi + 1, buf ^ 1)

        wait_gather(buf)

        @pl.when(qi > 0)
        def _():
            drain_writes()

        rows = rows_v.at[buf]

        def shuffle_body(i, carry):
            jr = lax.shift_right_logical(i, 3)
            dg = lax.bitwise_and(i, 7)
            t = dg * 16
            row0 = iota + t
            off = jr * 128 + t
            for b in range(8):
                row_b = row0 + (b * 128)
                for jt in range(4):
                    col = jnp.full((16,), jt * 8, jnp.int32) + jr
                    v = plsc.load_gather(rows, [row_b, col])
                    stage_v[b, jt, pl.ds(off, 16)] = v * _SCALE
            return carry

        lax.fori_loop(0, 64, shuffle_body, 0)

        for b in range(8):
            pltpu.make_async_copy(
                stage_v.at[b],
                out_hbm.at[a * 8 + b, :, pl.ds(c * _CHUNK, _CHUNK)],
                wsem,
            ).start()

    start_gather(0, 0)

    def pair_body(g, carry):
        do_chunk(g * 2, 0)
        do_chunk(g * 2 + 1, 1)
        return carry

    lax.fori_loop(0, _CPW // 2, pair_body, 0)
    do_chunk(_CPW - 1, 0)
    drain_writes()


@jax.jit
def _lookup(tok_view, table):
    mesh = plsc.VectorSubcoreMesh(core_axis_name="c", subcore_axis_name="s")
    run = functools.partial(
        pl.kernel,
        mesh=mesh,
        out_type=jax.ShapeDtypeStruct((_B1, 4, 32 * _CHUNK), jnp.float32),
        scratch_types=[
            pltpu.VMEM((2, _CHUNK), jnp.int32),
            pltpu.VMEM((2, _CHUNK, _EMB), jnp.float32),
            pltpu.VMEM((8, 4, _CHUNK), jnp.float32),
            pltpu.SemaphoreType.DMA((2,)),
            pltpu.SemaphoreType.DMA,
        ],
        compiler_params=pltpu.CompilerParams(
            use_tc_tiling_on_sc=False, needs_layout_passes=False),
    )(_emb_kernel)
    return run(tok_view, table)


def kernel(tokens, table):
    tok_view = (
        tokens.astype(jnp.int32)
        .T.reshape(25, 8, 32, 128)
        .transpose(0, 2, 1, 3)
        .reshape(_NQ, _CHUNK)
    )
    out4 = _lookup(tok_view, table)
    return (
        out4.reshape(_B1, 4, 32, 8, 128)
        .transpose(2, 4, 0, 1, 3)
        .reshape(_B0, _B1, _EMB)
    )


# speedup vs baseline: 1.3663x; 1.3663x over previous
import functools
import math

import jax
import jax.numpy as jnp
from jax import lax
from jax.experimental import pallas as pl
from jax.experimental.pallas import tpu as pltpu
from jax.experimental.pallas import tpu_sc as plsc

_EMB = 32
_SCALE = math.sqrt(_EMB)

_NC = 2
_NS = 16
_NW = _NC * _NS

_B0 = 4096
_B1 = 200
_CHUNK = 1024
_NQ = (_B0 // 128) * (_B1 // 8)
_CPW = _NQ // _NW


def _emb_kernel(tok_hbm, table_hbm, out_hbm, idx_v, rows_v, stage_v, gsem, wsem):
    wid = lax.axis_index("s") * _NC + lax.axis_index("c")
    q0 = wid * _CPW
    iota = lax.iota(jnp.int32, 16)

    def start_gather(qi, buf):
        pltpu.sync_copy(tok_hbm.at[q0 + qi], idx_v.at[buf])
        pltpu.async_copy(table_hbm.at[idx_v.at[buf]], rows_v.at[buf], gsem.at[buf])

    def wait_gather(buf):
        pltpu.make_async_copy(
            table_hbm.at[idx_v.at[buf]], rows_v.at[buf], gsem.at[buf]).wait()

    def drain_writes():
        for b in range(8):
            pltpu.make_async_copy(
                stage_v.at[b], out_hbm.at[0, :, pl.ds(0, _CHUNK)], wsem).wait()

    def do_chunk(qi, buf):
        q = q0 + qi
        a = lax.div(q, 32)
        c = lax.rem(q, 32)

        @pl.when(qi + 1 < _CPW)
        def _():
            start_gather(qi + 1, buf ^ 1)

        wait_gather(buf)

        @pl.when(qi > 0)
        def _():
            drain_writes()

        rows = rows_v.at[buf]

        @plsc.parallel_loop(0, 64)
        def _shuffle(i):
            b = lax.shift_right_logical(i, 3)
            dg = lax.bitwise_and(i, 7)
            t = dg * 16
            row = iota + (b * 128 + t)
            for jt in range(4):
                for jr in range(8):
                    col = jnp.full((16,), jt * 8 + jr, jnp.int32)
                    v = plsc.load_gather(rows, [row, col])
                    stage_v[b, jt, pl.ds(jr * 128 + t, 16)] = v * _SCALE

        for b in range(8):
            pltpu.make_async_copy(
                stage_v.at[b],
                out_hbm.at[a * 8 + b, :, pl.ds(c * _CHUNK, _CHUNK)],
                wsem,
            ).start()

    start_gather(0, 0)

    def pair_body(g, carry):
        do_chunk(g * 2, 0)
        do_chunk(g * 2 + 1, 1)
        return carry

    lax.fori_loop(0, _CPW // 2, pair_body, 0)
    do_chunk(_CPW - 1, 0)
    drain_writes()


@jax.jit
def _lookup(tok_view, table):
    mesh = plsc.VectorSubcoreMesh(core_axis_name="c", subcore_axis_name="s")
    run = functools.partial(
        pl.kernel,
        mesh=mesh,
        out_type=jax.ShapeDtypeStruct((_B1, 4, 32 * _CHUNK), jnp.float32),
        scratch_types=[
            pltpu.VMEM((2, _CHUNK), jnp.int32),
            pltpu.VMEM((2, _CHUNK, _EMB), jnp.float32),
            pltpu.VMEM((8, 4, _CHUNK), jnp.float32),
            pltpu.SemaphoreType.DMA((2,)),
            pltpu.SemaphoreType.DMA,
        ],
        compiler_params=pltpu.CompilerParams(
            use_tc_tiling_on_sc=False, needs_layout_passes=False),
    )(_emb_kernel)
    return run(tok_view, table)


def kernel(tokens, table):
    tok_view = (
        tokens.astype(jnp.int32)
        .T.reshape(25, 8, 32, 128)
        .transpose(0, 2, 1, 3)
        .reshape(_NQ, _CHUNK)
    )
    out4 = _lookup(tok_view, table)
    return (
        out4.reshape(_B1, 4, 32, 8, 128)
        .transpose(2, 4, 0, 1, 3)
        .reshape(_B0, _B1, _EMB)
    )
